# baseline (speedup 1.0000x reference)
import functools
import math

import jax
import jax.numpy as jnp
from jax import lax
from jax.experimental import pallas as pl
from jax.experimental.pallas import tpu as pltpu
from jax.experimental.pallas import tpu_sc as plsc

_NC = 2
_NS = 16
_NW = _NC * _NS


@functools.cache
def _build_gather(B, V, D, C):
  assert B % (_NW * C) == 0 and C % 8 == 0
  bpw = B // _NW
  T = bpw // C
  assert T % 2 == 0

  mesh = plsc.VectorSubcoreMesh(core_axis_name="c", subcore_axis_name="s")

  @functools.partial(
      pl.kernel,
      out_type=jax.ShapeDtypeStruct((B, 2 * D), jnp.float32),
      mesh=mesh,
      scratch_types=[
          pltpu.VMEM((bpw,), jnp.int32),
          pltpu.VMEM((C,), jnp.int32),
          pltpu.VMEM((C,), jnp.int32),
          pltpu.VMEM((2, C, 2 * D), jnp.float32),
          pltpu.SemaphoreType.DMA,
          pltpu.SemaphoreType.DMA,
          pltpu.SemaphoreType.DMA,
          pltpu.SemaphoreType.DMA,
      ],
      compiler_params=pltpu.CompilerParams(
          use_tc_tiling_on_sc=True, needs_layout_passes=False),
  )
  def gather_kernel(idx_hbm, table_hbm, out_hbm, idx_v, u_v0, u_v1, rows_v,
                    g0, g1, w0, w1):
    wid = lax.axis_index("s") * _NC + lax.axis_index("c")
    row0 = wid * bpw
    u_v = [u_v0, u_v1]
    gsem = [g0, g1]
    wsem = [w0, w1]

    pltpu.sync_copy(idx_hbm.at[pl.ds(row0, bpw)], idx_v)

    def fire_gather(t, b):
      for k in range(C // 16):
        sl = pl.ds(k * 16, 16)
        u_v[b][sl] = idx_v[pl.ds(t * C + k * 16, 16)] >> 1
      pltpu.async_copy(table_hbm.at[u_v[b]], rows_v.at[b], gsem[b])

    def wait_gather(b):
      pltpu.make_async_copy(table_hbm.at[u_v[b]], rows_v.at[b],
                            gsem[b]).wait()

    def fire_write(t, b):
      pltpu.async_copy(rows_v.at[b], out_hbm.at[pl.ds(row0 + t * C, C)],
                       wsem[b])

    def wait_write(t, b):
      pltpu.make_async_copy(rows_v.at[b], out_hbm.at[pl.ds(row0 + t * C, C)],
                            wsem[b]).wait()

    fire_gather(0, 0)

    def step(tt):
      for par in range(2):
        t = tt + par

        @pl.when(t + 1 < T)
        def _():
          fire_gather(t + 1, 1 - par)

        wait_gather(par)

        @pl.when(t >= 2)
        def _():
          wait_write(t - 2, par)

        fire_write(t, par)

    pl.loop(0, T, step=2)(step)

    wait_write(T - 2, 0)
    wait_write(T - 1, 1)

  return gather_kernel


@functools.cache
def _build_pack(B, S, D):
  Bb = B // S
  JB = Bb // 128
  scale = math.sqrt(D)

  def pack_kernel(rows_ref, par_ref, u_ref):
    blk = rows_ref[0]
    p = par_ref[0, 0][:, None]
    sel = jnp.where(p == 1, blk[:, D:], blk[:, :D])
    u_ref[0] = jnp.swapaxes(sel, 0, 1) * scale

  return pl.pallas_call(
      pack_kernel,
      grid=(S, JB),
      in_specs=[
          pl.BlockSpec((1, 128, 2 * D), lambda s, j: (s * JB + j, 0, 0)),
          pl.BlockSpec((1, 1, 128), lambda s, j: (s * JB + j, 0, 0)),
      ],
      out_specs=pl.BlockSpec((1, D, 128), lambda s, j: (s, 0, j)),
      out_shape=jax.ShapeDtypeStruct((S, D, Bb), jnp.float32),
  )


def kernel(x, table):
  Bb, S = x.shape
  V, D = table.shape
  B = Bb * S
  xf = jnp.transpose(x).reshape(B).astype(jnp.int32)
  t2 = table.reshape(V // 2, 2 * D)
  pairs = _build_gather(B, V, D, 256)(xf, t2)
  par3 = (xf & 1).reshape(B // 128, 1, 128)
  u = _build_pack(B, S, D)(pairs.reshape(B // 128, 128, 2 * D), par3)
  return jnp.transpose(u, (2, 0, 1))


# speedup vs baseline: 1.0002x; 1.0002x over previous
import functools
import math

import jax
import jax.numpy as jnp
from jax import lax
from jax.experimental import pallas as pl
from jax.experimental.pallas import tpu as pltpu
from jax.experimental.pallas import tpu_sc as plsc

_NC = 2
_NS = 16
_NW = _NC * _NS


@functools.cache
def _build_gather(B, V, D, C):
  assert B % (_NW * C) == 0 and C % 8 == 0
  bpw = B // _NW
  T = bpw // C
  assert T % 2 == 0

  mesh = plsc.VectorSubcoreMesh(core_axis_name="c", subcore_axis_name="s")

  @functools.partial(
      pl.kernel,
      out_type=jax.ShapeDtypeStruct((B, 2 * D), jnp.float32),
      mesh=mesh,
      scratch_types=[
          pltpu.VMEM((bpw,), jnp.int32),
          pltpu.VMEM((C,), jnp.int32),
          pltpu.VMEM((C,), jnp.int32),
          pltpu.VMEM((2, C, 2 * D), jnp.float32),
          pltpu.SemaphoreType.DMA,
          pltpu.SemaphoreType.DMA,
          pltpu.SemaphoreType.DMA,
          pltpu.SemaphoreType.DMA,
      ],
      compiler_params=pltpu.CompilerParams(
          use_tc_tiling_on_sc=True, needs_layout_passes=False),
  )
  def gather_kernel(idx_hbm, table_hbm, out_hbm, idx_v, u_v0, u_v1, rows_v,
                    g0, g1, w0, w1):
    wid = lax.axis_index("s") * _NC + lax.axis_index("c")
    row0 = wid * bpw
    u_v = [u_v0, u_v1]
    gsem = [g0, g1]
    wsem = [w0, w1]

    pltpu.sync_copy(idx_hbm.at[pl.ds(row0, bpw)], idx_v)

    def fire_gather(t, b):
      for k in range(C // 16):
        sl = pl.ds(k * 16, 16)
        u_v[b][sl] = idx_v[pl.ds(t * C + k * 16, 16)] >> 1
      pltpu.async_copy(table_hbm.at[u_v[b]], rows_v.at[b], gsem[b])

    def wait_gather(b):
      pltpu.make_async_copy(table_hbm.at[u_v[b]], rows_v.at[b],
                            gsem[b]).wait()

    def fire_write(t, b):
      pltpu.async_copy(rows_v.at[b], out_hbm.at[pl.ds(row0 + t * C, C)],
                       wsem[b])

    def wait_write(t, b):
      pltpu.make_async_copy(rows_v.at[b], out_hbm.at[pl.ds(row0 + t * C, C)],
                            wsem[b]).wait()

    fire_gather(0, 0)

    def step(tt):
      for par in range(2):
        t = tt + par

        @pl.when(t >= 1)
        def _():
          wait_write(t - 1, 1 - par)

        @pl.when(t + 1 < T)
        def _():
          fire_gather(t + 1, 1 - par)

        wait_gather(par)
        fire_write(t, par)

    pl.loop(0, T, step=2)(step)

    wait_write(T - 1, 1)

  return gather_kernel


@functools.cache
def _build_pack(B, S, D):
  Bb = B // S
  JB = Bb // 128
  scale = math.sqrt(D)

  def pack_kernel(rows_ref, par_ref, u_ref):
    blk = rows_ref[0]
    p = par_ref[0, 0][:, None]
    sel = jnp.where(p == 1, blk[:, D:], blk[:, :D])
    u_ref[0] = jnp.swapaxes(sel, 0, 1) * scale

  return pl.pallas_call(
      pack_kernel,
      grid=(S, JB),
      in_specs=[
          pl.BlockSpec((1, 128, 2 * D), lambda s, j: (s * JB + j, 0, 0)),
          pl.BlockSpec((1, 1, 128), lambda s, j: (s * JB + j, 0, 0)),
      ],
      out_specs=pl.BlockSpec((1, D, 128), lambda s, j: (s, 0, j)),
      out_shape=jax.ShapeDtypeStruct((S, D, Bb), jnp.float32),
  )


def kernel(x, table):
  Bb, S = x.shape
  V, D = table.shape
  B = Bb * S
  xf = jnp.transpose(x).reshape(B).astype(jnp.int32)
  t2 = table.reshape(V // 2, 2 * D)
  pairs = _build_gather(B, V, D, 256)(xf, t2)
  par3 = (xf & 1).reshape(B // 128, 1, 128)
  u = _build_pack(B, S, D)(pairs.reshape(B // 128, 128, 2 * D), par3)
  return jnp.transpose(u, (2, 0, 1))
